# speedup vs baseline: 5.2246x; 5.2246x over previous
import functools

import jax
import jax.numpy as jnp
from jax import lax
from jax.experimental import pallas as pl
from jax.experimental.pallas import tpu as pltpu
from jax.experimental.pallas import tpu_sc as plsc

B = 16384
L = 20
E = 128
N = 1000

NC = 2
NS = 16
NW = NC * NS
ROWS_PER_W = B // NW
CHUNK_ROWS = 4
CHUNK_IDX = CHUNK_ROWS * L
NCHUNKS = ROWS_PER_W // CHUNK_ROWS
EV = E // 16


def _pool_sc(xr, table):
    mesh = plsc.VectorSubcoreMesh(core_axis_name="c", subcore_axis_name="s")

    @functools.partial(
        pl.kernel,
        mesh=mesh,
        out_type=jax.ShapeDtypeStruct((B, E), jnp.float32),
        scratch_types=[
            pltpu.VMEM((NCHUNKS, CHUNK_IDX), jnp.int32),
            pltpu.VMEM((CHUNK_IDX, E), jnp.float32),
            pltpu.VMEM((CHUNK_IDX, E), jnp.float32),
            pltpu.VMEM((ROWS_PER_W, E), jnp.float32),
            pltpu.SemaphoreType.DMA,
            pltpu.SemaphoreType.DMA,
        ],
    )
    def pool(x_hbm, table_hbm, out_hbm, idx_v, buf0, buf1, out_v, sem0, sem1):
        wid = lax.axis_index("s") * NC + lax.axis_index("c")

        pltpu.sync_copy(x_hbm.at[pl.ds(wid * NCHUNKS, NCHUNKS)], idx_v)

        def fire(c, buf, sem):
            pltpu.async_copy(table_hbm.at[idx_v.at[c]], buf, sem)

        def drain(buf, sem):
            pltpu.make_async_copy(table_hbm.at[pl.ds(0, CHUNK_IDX)], buf, sem).wait()

        def accumulate(buf, c):
            for rr in range(CHUNK_ROWS):
                acc = [buf[rr * L, pl.ds(e * 16, 16)] for e in range(EV)]
                for l in range(1, L):
                    for e in range(EV):
                        acc[e] = acc[e] + buf[rr * L + l, pl.ds(e * 16, 16)]
                row = c * CHUNK_ROWS + rr
                for e in range(EV):
                    out_v[row, pl.ds(e * 16, 16)] = acc[e]

        fire(0, buf0, sem0)
        fire(1, buf1, sem1)

        def body(c2, carry):
            c0 = c2 * 2
            drain(buf0, sem0)
            accumulate(buf0, c0)

            @pl.when(c2 < NCHUNKS // 2 - 1)
            def _():
                fire(c0 + 2, buf0, sem0)

            drain(buf1, sem1)
            accumulate(buf1, c0 + 1)

            @pl.when(c2 < NCHUNKS // 2 - 1)
            def _():
                fire(c0 + 3, buf1, sem1)

            return carry

        lax.fori_loop(0, NCHUNKS // 2, body, 0)

        pltpu.sync_copy(out_v, out_hbm.at[pl.ds(wid * ROWS_PER_W, ROWS_PER_W)])

    return pool(xr, table)


def _mm_body(s_ref, x_ref, w_ref, b_ref, o_ref):
    cnt = jnp.sum((x_ref[...] != 0).astype(jnp.float32), axis=1, keepdims=True)
    denom = jnp.maximum(cnt, 1.0)
    mean = s_ref[...] / denom
    o_ref[...] = (
        jnp.dot(mean, w_ref[...], preferred_element_type=jnp.float32) + b_ref[...]
    )


def _matmul_tc(summed, x32, fc_w, fc_b2):
    BM = 1024
    return pl.pallas_call(
        _mm_body,
        grid=(B // BM,),
        in_specs=[
            pl.BlockSpec((BM, E), lambda i: (i, 0)),
            pl.BlockSpec((BM, L), lambda i: (i, 0)),
            pl.BlockSpec((E, N), lambda i: (0, 0)),
            pl.BlockSpec((1, N), lambda i: (0, 0)),
        ],
        out_specs=pl.BlockSpec((BM, N), lambda i: (i, 0)),
        out_shape=jax.ShapeDtypeStruct((B, N), jnp.float32),
    )(summed, x32, fc_w, fc_b2)


def kernel(x, emb_table, fc_w, fc_b):
    x32 = x.astype(jnp.int32)
    xr = x32.reshape(B * L // CHUNK_IDX, CHUNK_IDX)
    summed = _pool_sc(xr, emb_table)
    return _matmul_tc(summed, x32, fc_w, fc_b.reshape(1, N))
